# Initial kernel scaffold; baseline (speedup 1.0000x reference)
import jax
import jax.numpy as jnp
from jax.experimental import pallas as pl


def kernel(user_emb, item_emb, W_gc_0, b_gc_0, W_bi_0, b_bi_0, W_gc_1, b_gc_1, W_bi_1, b_bi_1, W_gc_2, b_gc_2, W_bi_2, b_bi_2, adj_vals, adj_idx, users, pos_items, neg_items):
    raise NotImplementedError("write your pallas kernel here")


# speedup vs baseline: 2.3595x; 2.3595x over previous
import functools

import jax
import jax.numpy as jnp
from jax import lax
from jax.experimental import pallas as pl
from jax.experimental.pallas import tpu as pltpu
from jax.experimental.pallas import tpu_sc as plsc

N_USER = 25000
N_ITEM = 25000
N = N_USER + N_ITEM
D = 64
NNZ = 800000
LANES = 16

NC = 2
NS = 16
NW = NC * NS

NP = 50176
RPC = NP // NC
RPT = RPC // NS

NNZ_P = 804864
EPT = NNZ_P // NS
CHUNK = 128
GC = 3
GROUP = CHUNK * GC
NGROUPS = EPT // GROUP

TC_BLK = 512
TC_GRID = NP // TC_BLK

GB = 3 * 1024
GPW = GB // NW


def _spmm_body(ego, row1, col1, val1, zrows, out,
               rowv, colv, valv, idx0, idx1, idx2, gbuf, sem, acc):
    core = lax.axis_index("c")
    sub = lax.axis_index("s")
    idxrefs = [idx0, idx1, idx2]

    pltpu.sync_copy(zrows, acc.at[pl.ds(sub * RPT, RPT)])
    plsc.subcore_barrier()

    row_base = core * RPC

    def group_body(g, _):
        eb = sub * EPT + g * GROUP
        c1 = pltpu.async_copy(row1.at[pl.ds(eb, GROUP)], rowv, sem)
        c2 = pltpu.async_copy(col1.at[pl.ds(eb, GROUP)], colv, sem)
        c3 = pltpu.async_copy(val1.at[pl.ds(eb, GROUP)], valv, sem)
        c1.wait(); c2.wait(); c3.wait()

        for c in range(GC):
            def prep(j, _):
                sl = pl.ds(c * CHUNK + j * LANES, LANES)
                lsl = pl.ds(j * LANES, LANES)
                lr = rowv[sl] - row_base
                owned = (lr >= 0) & (lr < RPC)
                idxrefs[c][lsl] = jnp.where(owned, lr, 0)
                valv[sl] = jnp.where(owned, valv[sl], 0.0)
                return 0
            lax.fori_loop(0, CHUNK // LANES, prep, 0, unroll=False)

        waits = [
            pltpu.async_copy(ego.at[colv.at[pl.ds(c * CHUNK, CHUNK)]],
                             gbuf.at[pl.ds(c * CHUNK, CHUNK)], sem)
            for c in range(GC)
        ]
        for w in waits:
            w.wait()

        def scale(i, _):
            vv = valv[pl.ds(i * LANES, LANES)]
            for k in range(LANES):
                e = i * LANES + k
                vs = jnp.full((LANES,), vv[k], jnp.float32)
                for q in range(D // LANES):
                    sl = pl.ds(q * LANES, LANES)
                    gbuf[e, sl] = gbuf[e, sl] * vs
            return 0
        lax.fori_loop(0, GROUP // LANES, scale, 0, unroll=False)

        for c in range(GC):
            pltpu.sync_copy(gbuf.at[pl.ds(c * CHUNK, CHUNK)],
                            acc.at[idxrefs[c]], add=True)
        return 0

    lax.fori_loop(0, NGROUPS, group_body, 0, unroll=False)

    plsc.subcore_barrier()
    pltpu.sync_copy(acc.at[pl.ds(sub * RPT, RPT)],
                    out.at[pl.ds(core * RPC + sub * RPT, RPT)])


@jax.jit
def _spmm(ego, row1, col1, val1):
    zrows = jnp.zeros((RPT, D), jnp.float32)
    mesh = plsc.VectorSubcoreMesh(core_axis_name="c", subcore_axis_name="s")
    return pl.kernel(
        _spmm_body,
        out_type=jax.ShapeDtypeStruct((NP, D), jnp.float32),
        mesh=mesh,
        scratch_types=[
            pltpu.VMEM((GROUP,), jnp.int32),
            pltpu.VMEM((GROUP,), jnp.int32),
            pltpu.VMEM((GROUP,), jnp.float32),
            pltpu.VMEM((CHUNK,), jnp.int32),
            pltpu.VMEM((CHUNK,), jnp.int32),
            pltpu.VMEM((CHUNK,), jnp.int32),
            pltpu.VMEM((GROUP, D), jnp.float32),
            pltpu.SemaphoreType.DMA,
            pltpu.VMEM_SHARED((RPC, D), jnp.float32),
        ],
        compiler_params=pltpu.CompilerParams(use_tc_tiling_on_sc=False),
    )(ego, row1, col1, val1, zrows)


def _dense_body(ego, side, wgc, bgc, wbi, bbi, new_ego, norm):
    e = ego[...]
    s = side[...]
    x = (jnp.dot(s, wgc[...], preferred_element_type=jnp.float32) + bgc[...]
         + jnp.dot(e * s, wbi[...], preferred_element_type=jnp.float32)
         + bbi[...])
    h = jnp.where(x >= 0, x, 0.2 * x)
    n2 = jnp.sum(h * h, axis=1, keepdims=True)
    new_ego[...] = h
    norm[...] = h / jnp.maximum(jnp.sqrt(n2), 1e-12)


@jax.jit
def _dense(ego, side, wgc, bgc, wbi, bbi):
    blk = pl.BlockSpec((TC_BLK, D), lambda i: (i, 0))
    full = pl.BlockSpec((D, D), lambda i: (0, 0))
    bias = pl.BlockSpec((1, D), lambda i: (0, 0))
    return pl.pallas_call(
        _dense_body,
        grid=(TC_GRID,),
        in_specs=[blk, blk, full, bias, full, bias],
        out_specs=[blk, blk],
        out_shape=[jax.ShapeDtypeStruct((NP, D), jnp.float32),
                   jax.ShapeDtypeStruct((NP, D), jnp.float32)],
    )(ego, side, wgc, bgc, wbi, bbi)


def _lookup_body(s0, s1, s2, s3, idx, o0, o1, o2, o3, idxv, rbuf, sem):
    core = lax.axis_index("c")
    sub = lax.axis_index("s")
    wid = sub * NC + core
    base = wid * GPW
    pltpu.sync_copy(idx.at[pl.ds(base, GPW)], idxv)
    for slab, o in ((s0, o0), (s1, o1), (s2, o2), (s3, o3)):
        pltpu.async_copy(slab.at[idxv], rbuf, sem).wait()
        pltpu.sync_copy(rbuf, o.at[pl.ds(base, GPW)])


@jax.jit
def _lookup(s0, s1, s2, s3, idx):
    mesh = plsc.VectorSubcoreMesh(core_axis_name="c", subcore_axis_name="s")
    out = jax.ShapeDtypeStruct((GB, D), jnp.float32)
    return pl.kernel(
        _lookup_body,
        out_type=(out, out, out, out),
        mesh=mesh,
        scratch_types=[
            pltpu.VMEM((GPW,), jnp.int32),
            pltpu.VMEM((GPW, D), jnp.float32),
            pltpu.SemaphoreType.DMA,
        ],
        compiler_params=pltpu.CompilerParams(use_tc_tiling_on_sc=False),
    )(s0, s1, s2, s3, idx)


def kernel(user_emb, item_emb,
           W_gc_0, b_gc_0, W_bi_0, b_bi_0,
           W_gc_1, b_gc_1, W_bi_1, b_bi_1,
           W_gc_2, b_gc_2, W_bi_2, b_bi_2,
           adj_vals, adj_idx, users, pos_items, neg_items):
    Ws = [(W_gc_0, b_gc_0, W_bi_0, b_bi_0),
          (W_gc_1, b_gc_1, W_bi_1, b_bi_1),
          (W_gc_2, b_gc_2, W_bi_2, b_bi_2)]

    ego0 = jnp.zeros((NP, D), jnp.float32)
    ego0 = ego0.at[:N_USER].set(user_emb).at[N_USER:N].set(item_emb)

    row = jnp.pad(adj_idx[0].astype(jnp.int32), (0, NNZ_P - NNZ))
    col = jnp.pad(adj_idx[1].astype(jnp.int32), (0, NNZ_P - NNZ))
    val = jnp.pad(adj_vals.astype(jnp.float32), (0, NNZ_P - NNZ))

    slabs = [ego0]
    ego = ego0
    for (wgc, bgc, wbi, bbi) in Ws:
        side = _spmm(ego, row, col, val)
        ego, nrm = _dense(ego, side, wgc, bgc, wbi, bbi)
        slabs.append(nrm)

    idx = jnp.concatenate([users.astype(jnp.int32),
                           pos_items.astype(jnp.int32) + N_USER,
                           neg_items.astype(jnp.int32) + N_USER])
    g0, g1, g2, g3 = _lookup(slabs[0], slabs[1], slabs[2], slabs[3], idx)
    all_e = jnp.concatenate([g0, g1, g2, g3], axis=1)
    B = users.shape[0]
    return (all_e[:B], all_e[B:2 * B], all_e[2 * B:])


# speedup vs baseline: 5.2536x; 2.2265x over previous
import functools

import jax
import jax.numpy as jnp
from jax import lax
from jax.experimental import pallas as pl
from jax.experimental.pallas import tpu as pltpu
from jax.experimental.pallas import tpu_sc as plsc

N_USER = 25000
N_ITEM = 25000
N = N_USER + N_ITEM
D = 64
DH = D // 2
NNZ = 800000
LANES = 16

NC = 2
NS = 16
NW = NC * NS

NP = 50176
ZPT = NP // NS

NNZ_P = 804864
EPT = NNZ_P // NS
CHUNK = 128
GC = 3
GROUP = CHUNK * GC
NGROUPS = EPT // GROUP

TC_BLK = 512
TC_GRID = NP // TC_BLK

GB = 3 * 1024
GPW = GB // NW


def _spmm_body(ego_lo, ego_hi, row2, col1, val1, zrows, out_lo, out_hi,
               colv, valv, idx0, idx1, idx2, gbuf, sem, acc):
    core = lax.axis_index("c")
    sub = lax.axis_index("s")
    idxrefs = [idx0, idx1, idx2]

    pltpu.sync_copy(zrows, acc.at[pl.ds(sub * ZPT, ZPT)])
    plsc.subcore_barrier()

    def run(ego, out):
        def group_body(g, _):
            eb = sub * EPT + g * GROUP
            cb = sub * (EPT // CHUNK) + g * GC
            cps = [pltpu.async_copy(col1.at[pl.ds(eb, GROUP)], colv, sem),
                   pltpu.async_copy(val1.at[pl.ds(eb, GROUP)], valv, sem)]
            cps += [pltpu.async_copy(row2.at[cb + c], idxrefs[c], sem)
                    for c in range(GC)]
            for cp in cps:
                cp.wait()

            waits = [
                pltpu.async_copy(ego.at[colv.at[pl.ds(c * CHUNK, CHUNK)]],
                                 gbuf.at[pl.ds(c * CHUNK, CHUNK)], sem)
                for c in range(GC)
            ]
            for w in waits:
                w.wait()

            def scale(i, _):
                vv = valv[pl.ds(i * LANES, LANES)]
                for k in range(LANES):
                    e = i * LANES + k
                    vs = jnp.full((LANES,), vv[k], jnp.float32)
                    for q in range(DH // LANES):
                        sl = pl.ds(q * LANES, LANES)
                        gbuf[e, sl] = gbuf[e, sl] * vs
                return 0
            lax.fori_loop(0, GROUP // LANES, scale, 0, unroll=False)

            for c in range(GC):
                pltpu.sync_copy(gbuf.at[pl.ds(c * CHUNK, CHUNK)],
                                acc.at[idxrefs[c]], add=True)
            return 0

        lax.fori_loop(0, NGROUPS, group_body, 0, unroll=False)

        plsc.subcore_barrier()
        pltpu.sync_copy(acc.at[pl.ds(sub * ZPT, ZPT)],
                        out.at[pl.ds(sub * ZPT, ZPT)])

    @pl.when(core == 0)
    def _():
        run(ego_lo, out_lo)

    @pl.when(core == 1)
    def _():
        run(ego_hi, out_hi)


@jax.jit
def _spmm(ego_lo, ego_hi, row2, col1, val1):
    zrows = jnp.zeros((ZPT, DH), jnp.float32)
    mesh = plsc.VectorSubcoreMesh(core_axis_name="c", subcore_axis_name="s")
    half = jax.ShapeDtypeStruct((NP, DH), jnp.float32)
    return pl.kernel(
        _spmm_body,
        out_type=(half, half),
        mesh=mesh,
        scratch_types=[
            pltpu.VMEM((GROUP,), jnp.int32),
            pltpu.VMEM((GROUP,), jnp.float32),
            pltpu.VMEM((CHUNK,), jnp.int32),
            pltpu.VMEM((CHUNK,), jnp.int32),
            pltpu.VMEM((CHUNK,), jnp.int32),
            pltpu.VMEM((GROUP, DH), jnp.float32),
            pltpu.SemaphoreType.DMA,
            pltpu.VMEM_SHARED((NP, DH), jnp.float32),
        ],
        compiler_params=pltpu.CompilerParams(use_tc_tiling_on_sc=False),
    )(ego_lo, ego_hi, row2, col1, val1, zrows)


def _dense_body(elo, ehi, slo, shi, wgc, bgc, wbi, bbi, new_lo, new_hi, norm):
    e = jnp.concatenate([elo[...], ehi[...]], axis=1)
    s = jnp.concatenate([slo[...], shi[...]], axis=1)
    x = (jnp.dot(s, wgc[...], preferred_element_type=jnp.float32) + bgc[...]
         + jnp.dot(e * s, wbi[...], preferred_element_type=jnp.float32)
         + bbi[...])
    h = jnp.where(x >= 0, x, 0.2 * x)
    n2 = jnp.sum(h * h, axis=1, keepdims=True)
    new_lo[...] = h[:, :DH]
    new_hi[...] = h[:, DH:]
    norm[...] = h / jnp.maximum(jnp.sqrt(n2), 1e-12)


@jax.jit
def _dense(elo, ehi, slo, shi, wgc, bgc, wbi, bbi):
    blk_h = pl.BlockSpec((TC_BLK, DH), lambda i: (i, 0))
    blk_f = pl.BlockSpec((TC_BLK, D), lambda i: (i, 0))
    full = pl.BlockSpec((D, D), lambda i: (0, 0))
    bias = pl.BlockSpec((1, D), lambda i: (0, 0))
    half = jax.ShapeDtypeStruct((NP, DH), jnp.float32)
    return pl.pallas_call(
        _dense_body,
        grid=(TC_GRID,),
        in_specs=[blk_h, blk_h, blk_h, blk_h, full, bias, full, bias],
        out_specs=[blk_h, blk_h, blk_f],
        out_shape=[half, half, jax.ShapeDtypeStruct((NP, D), jnp.float32)],
    )(elo, ehi, slo, shi, wgc, bgc, wbi, bbi)


def _lookup_body(s0, s1, s2, s3, idx, o0, o1, o2, o3, idxv, rbuf, sem):
    core = lax.axis_index("c")
    sub = lax.axis_index("s")
    wid = sub * NC + core
    base = wid * GPW
    pltpu.sync_copy(idx.at[pl.ds(base, GPW)], idxv)
    for slab, o in ((s0, o0), (s1, o1), (s2, o2), (s3, o3)):
        pltpu.async_copy(slab.at[idxv], rbuf, sem).wait()
        pltpu.sync_copy(rbuf, o.at[pl.ds(base, GPW)])


@jax.jit
def _lookup(s0, s1, s2, s3, idx):
    mesh = plsc.VectorSubcoreMesh(core_axis_name="c", subcore_axis_name="s")
    out = jax.ShapeDtypeStruct((GB, D), jnp.float32)
    return pl.kernel(
        _lookup_body,
        out_type=(out, out, out, out),
        mesh=mesh,
        scratch_types=[
            pltpu.VMEM((GPW,), jnp.int32),
            pltpu.VMEM((GPW, D), jnp.float32),
            pltpu.SemaphoreType.DMA,
        ],
        compiler_params=pltpu.CompilerParams(use_tc_tiling_on_sc=False),
    )(s0, s1, s2, s3, idx)


def kernel(user_emb, item_emb,
           W_gc_0, b_gc_0, W_bi_0, b_bi_0,
           W_gc_1, b_gc_1, W_bi_1, b_bi_1,
           W_gc_2, b_gc_2, W_bi_2, b_bi_2,
           adj_vals, adj_idx, users, pos_items, neg_items):
    Ws = [(W_gc_0, b_gc_0, W_bi_0, b_bi_0),
          (W_gc_1, b_gc_1, W_bi_1, b_bi_1),
          (W_gc_2, b_gc_2, W_bi_2, b_bi_2)]

    ego0 = jnp.zeros((NP, D), jnp.float32)
    ego0 = ego0.at[:N_USER].set(user_emb).at[N_USER:N].set(item_emb)

    row = jnp.pad(adj_idx[0].astype(jnp.int32), (0, NNZ_P - NNZ))
    col = jnp.pad(adj_idx[1].astype(jnp.int32), (0, NNZ_P - NNZ))
    val = jnp.pad(adj_vals.astype(jnp.float32), (0, NNZ_P - NNZ))
    row2 = row.reshape(NNZ_P // CHUNK, CHUNK)

    slabs = [ego0]
    elo, ehi = ego0[:, :DH], ego0[:, DH:]
    for (wgc, bgc, wbi, bbi) in Ws:
        slo, shi = _spmm(elo, ehi, row2, col, val)
        elo, ehi, nrm = _dense(elo, ehi, slo, shi, wgc, bgc, wbi, bbi)
        slabs.append(nrm)

    idx = jnp.concatenate([users.astype(jnp.int32),
                           pos_items.astype(jnp.int32) + N_USER,
                           neg_items.astype(jnp.int32) + N_USER])
    g0, g1, g2, g3 = _lookup(slabs[0], slabs[1], slabs[2], slabs[3], idx)
    all_e = jnp.concatenate([g0, g1, g2, g3], axis=1)
    B = users.shape[0]
    return (all_e[:B], all_e[B:2 * B], all_e[2 * B:])


# speedup vs baseline: 6.2534x; 1.1903x over previous
import functools

import jax
import jax.numpy as jnp
from jax import lax
from jax.experimental import pallas as pl
from jax.experimental.pallas import tpu as pltpu
from jax.experimental.pallas import tpu_sc as plsc

N_USER = 25000
N_ITEM = 25000
N = N_USER + N_ITEM
D = 64
DH = D // 2
NNZ = 800000
LANES = 16

NC = 2
NS = 16
NW = NC * NS

NP = 50176
ZPT = NP // NS

NNZ_P = 811008
EPT = NNZ_P // NS
CHUNK = 128
GC = 3
GROUP = CHUNK * GC
NGROUPS = EPT // GROUP

TC_BLK = 512
TC_GRID = NP // TC_BLK

GB = 3 * 1024
GPW = GB // NW


def _spmm_body(ego_lo, ego_hi, meta3, zrows, out_lo, out_hi,
               m0, m1, m2, m3, gb0, gb1,
               semg0, semg1, sems0, sems1, semm0, semm1, acc):
    core = lax.axis_index("c")
    sub = lax.axis_index("s")
    metas = [m0, m1, m2, m3]
    gbufs = [gb0, gb1]
    semg = [semg0, semg1]
    sems = [sems0, sems1]
    semm = [semm0, semm1]

    pltpu.sync_copy(zrows, acc.at[pl.ds(sub * ZPT, ZPT)])
    plsc.subcore_barrier()

    cbase = sub * (EPT // CHUNK)

    def stage_meta(i, slot, sem):
        return pltpu.async_copy(meta3.at[pl.ds(cbase + i * GC, GC)],
                                metas[slot], sem)

    def issue_gathers(ego, i_slot, p):
        return [pltpu.async_copy(
            ego.at[metas[i_slot].at[c, 1]],
            gbufs[p].at[pl.ds(c * CHUNK, CHUNK)], semg[p])
            for c in range(GC)]

    def issue_scatters(i_slot, p):
        return [pltpu.async_copy(
            gbufs[p].at[pl.ds(c * CHUNK, CHUNK)],
            acc.at[metas[i_slot].at[c, 0]], sems[p], add=True)
            for c in range(GC)]

    def scale(mslot, p):
        gbuf = gbufs[p]
        mv = metas[mslot]

        def body(i, _):
            c = i // (CHUNK // LANES)
            o = (i % (CHUNK // LANES)) * LANES
            vv = lax.bitcast_convert_type(mv[c, 2, pl.ds(o, LANES)],
                                          jnp.float32)
            for k in range(LANES):
                e = i * LANES + k
                vs = jnp.full((LANES,), vv[k], jnp.float32)
                for q in range(DH // LANES):
                    sl = pl.ds(q * LANES, LANES)
                    gbuf[e, sl] = gbuf[e, sl] * vs
            return 0
        lax.fori_loop(0, GROUP // LANES, body, 0, unroll=False)

    def run(ego, out):
        stage_meta(0, 0, semm[0]).wait()
        cp1 = stage_meta(1, 1, semm[1])
        g_pro = issue_gathers(ego, 0, 0)

        def outer(g4, _):
            for j in range(4):
                i = g4 * 4 + j
                p, q = j % 2, 1 - (j % 2)
                mslot = j
                for c in range(GC):
                    pltpu.make_async_copy(
                        ego.at[metas[mslot].at[c, 1]],
                        gbufs[p].at[pl.ds(c * CHUNK, CHUNK)],
                        semg[p]).wait()
                @pl.when(i + 2 < NGROUPS)
                def _():
                    stage_meta(i + 2, (j + 2) % 4, semm[p])
                @pl.when(i + 1 < NGROUPS)
                def _():
                    pltpu.make_async_copy(
                        meta3.at[pl.ds(cbase + (i + 1) * GC, GC)],
                        metas[(j + 1) % 4], semm[q]).wait()

                @pl.when(i >= 1)
                def _():
                    for c in range(GC):
                        pltpu.make_async_copy(
                            gbufs[q].at[pl.ds(c * CHUNK, CHUNK)],
                            acc.at[metas[(j + 3) % 4].at[c, 0]],
                            sems[q]).wait()

                @pl.when(i + 1 < NGROUPS)
                def _():
                    issue_gathers(ego, (j + 1) % 4, q)

                scale(mslot, p)
                issue_scatters(mslot, p)
            return 0

        lax.fori_loop(0, NGROUPS // 4, outer, 0, unroll=False)

        for c in range(GC):
            pltpu.make_async_copy(
                gbufs[1].at[pl.ds(c * CHUNK, CHUNK)],
                acc.at[metas[3].at[c, 0]], sems[1]).wait()

        plsc.subcore_barrier()
        pltpu.sync_copy(acc.at[pl.ds(sub * ZPT, ZPT)],
                        out.at[pl.ds(sub * ZPT, ZPT)])

    @pl.when(core == 0)
    def _():
        run(ego_lo, out_lo)

    @pl.when(core == 1)
    def _():
        run(ego_hi, out_hi)


@jax.jit
def _spmm(ego_lo, ego_hi, meta3):
    zrows = jnp.zeros((ZPT, DH), jnp.float32)
    mesh = plsc.VectorSubcoreMesh(core_axis_name="c", subcore_axis_name="s")
    half = jax.ShapeDtypeStruct((NP, DH), jnp.float32)
    return pl.kernel(
        _spmm_body,
        out_type=(half, half),
        mesh=mesh,
        scratch_types=[
            pltpu.VMEM((GC, 3, CHUNK), jnp.int32),
            pltpu.VMEM((GC, 3, CHUNK), jnp.int32),
            pltpu.VMEM((GC, 3, CHUNK), jnp.int32),
            pltpu.VMEM((GC, 3, CHUNK), jnp.int32),
            pltpu.VMEM((GROUP, DH), jnp.float32),
            pltpu.VMEM((GROUP, DH), jnp.float32),
            pltpu.SemaphoreType.DMA,
            pltpu.SemaphoreType.DMA,
            pltpu.SemaphoreType.DMA,
            pltpu.SemaphoreType.DMA,
            pltpu.SemaphoreType.DMA,
            pltpu.SemaphoreType.DMA,
            pltpu.VMEM_SHARED((NP, DH), jnp.float32),
        ],
        compiler_params=pltpu.CompilerParams(use_tc_tiling_on_sc=False),
    )(ego_lo, ego_hi, meta3, zrows)


def _dense_body(elo, ehi, slo, shi, wgc, bgc, wbi, bbi, new_lo, new_hi, norm):
    e = jnp.concatenate([elo[...], ehi[...]], axis=1)
    s = jnp.concatenate([slo[...], shi[...]], axis=1)
    x = (jnp.dot(s, wgc[...], preferred_element_type=jnp.float32) + bgc[...]
         + jnp.dot(e * s, wbi[...], preferred_element_type=jnp.float32)
         + bbi[...])
    h = jnp.where(x >= 0, x, 0.2 * x)
    n2 = jnp.sum(h * h, axis=1, keepdims=True)
    new_lo[...] = h[:, :DH]
    new_hi[...] = h[:, DH:]
    norm[...] = h / jnp.maximum(jnp.sqrt(n2), 1e-12)


@jax.jit
def _dense(elo, ehi, slo, shi, wgc, bgc, wbi, bbi):
    blk_h = pl.BlockSpec((TC_BLK, DH), lambda i: (i, 0))
    blk_f = pl.BlockSpec((TC_BLK, D), lambda i: (i, 0))
    full = pl.BlockSpec((D, D), lambda i: (0, 0))
    bias = pl.BlockSpec((1, D), lambda i: (0, 0))
    half = jax.ShapeDtypeStruct((NP, DH), jnp.float32)
    return pl.pallas_call(
        _dense_body,
        grid=(TC_GRID,),
        in_specs=[blk_h, blk_h, blk_h, blk_h, full, bias, full, bias],
        out_specs=[blk_h, blk_h, blk_f],
        out_shape=[half, half, jax.ShapeDtypeStruct((NP, D), jnp.float32)],
    )(elo, ehi, slo, shi, wgc, bgc, wbi, bbi)


def _lookup_body(s0, s1, s2, s3, idx, o0, o1, o2, o3, idxv, rbuf, sem):
    core = lax.axis_index("c")
    sub = lax.axis_index("s")
    wid = sub * NC + core
    base = wid * GPW
    pltpu.sync_copy(idx.at[pl.ds(base, GPW)], idxv)
    for slab, o in ((s0, o0), (s1, o1), (s2, o2), (s3, o3)):
        pltpu.async_copy(slab.at[idxv], rbuf, sem).wait()
        pltpu.sync_copy(rbuf, o.at[pl.ds(base, GPW)])


@jax.jit
def _lookup(s0, s1, s2, s3, idx):
    mesh = plsc.VectorSubcoreMesh(core_axis_name="c", subcore_axis_name="s")
    out = jax.ShapeDtypeStruct((GB, D), jnp.float32)
    return pl.kernel(
        _lookup_body,
        out_type=(out, out, out, out),
        mesh=mesh,
        scratch_types=[
            pltpu.VMEM((GPW,), jnp.int32),
            pltpu.VMEM((GPW, D), jnp.float32),
            pltpu.SemaphoreType.DMA,
        ],
        compiler_params=pltpu.CompilerParams(use_tc_tiling_on_sc=False),
    )(s0, s1, s2, s3, idx)


def kernel(user_emb, item_emb,
           W_gc_0, b_gc_0, W_bi_0, b_bi_0,
           W_gc_1, b_gc_1, W_bi_1, b_bi_1,
           W_gc_2, b_gc_2, W_bi_2, b_bi_2,
           adj_vals, adj_idx, users, pos_items, neg_items):
    Ws = [(W_gc_0, b_gc_0, W_bi_0, b_bi_0),
          (W_gc_1, b_gc_1, W_bi_1, b_bi_1),
          (W_gc_2, b_gc_2, W_bi_2, b_bi_2)]

    ego0 = jnp.zeros((NP, D), jnp.float32)
    ego0 = ego0.at[:N_USER].set(user_emb).at[N_USER:N].set(item_emb)

    row = jnp.pad(adj_idx[0].astype(jnp.int32), (0, NNZ_P - NNZ))
    col = jnp.pad(adj_idx[1].astype(jnp.int32), (0, NNZ_P - NNZ))
    val = jnp.pad(adj_vals.astype(jnp.float32), (0, NNZ_P - NNZ))
    vbits = lax.bitcast_convert_type(val, jnp.int32)
    meta3 = jnp.stack([row.reshape(-1, CHUNK), col.reshape(-1, CHUNK),
                       vbits.reshape(-1, CHUNK)], axis=1)

    slabs = [ego0]
    elo, ehi = ego0[:, :DH], ego0[:, DH:]
    for (wgc, bgc, wbi, bbi) in Ws:
        slo, shi = _spmm(elo, ehi, meta3)
        elo, ehi, nrm = _dense(elo, ehi, slo, shi, wgc, bgc, wbi, bbi)
        slabs.append(nrm)

    idx = jnp.concatenate([users.astype(jnp.int32),
                           pos_items.astype(jnp.int32) + N_USER,
                           neg_items.astype(jnp.int32) + N_USER])
    g0, g1, g2, g3 = _lookup(slabs[0], slabs[1], slabs[2], slabs[3], idx)
    all_e = jnp.concatenate([g0, g1, g2, g3], axis=1)
    B = users.shape[0]
    return (all_e[:B], all_e[B:2 * B], all_e[2 * B:])


# speedup vs baseline: 6.4611x; 1.0332x over previous
import functools

import jax
import jax.numpy as jnp
from jax import lax
from jax.experimental import pallas as pl
from jax.experimental.pallas import tpu as pltpu
from jax.experimental.pallas import tpu_sc as plsc

N_USER = 25000
N_ITEM = 25000
N = N_USER + N_ITEM
D = 64
DH = D // 2
NNZ = 800000
LANES = 16

NC = 2
NS = 16
NW = NC * NS

NP = 50176
ZPT = NP // NS

NNZ_P = 811008
EPT = NNZ_P // NS
CHUNK = 128
GC = 3
GROUP = CHUNK * GC
NGROUPS = EPT // GROUP

TC_BLK = 512
TC_GRID = NP // TC_BLK

GB = 3 * 1024
GPW = GB // NW


def _spmm_body(ego_lo, ego_hi, adj3, val2, zrows, out_lo, out_hi,
               r0, r1, r2, r3, c0, c1, c2, c3, v0, v1, v2, v3, gb0, gb1,
               semg0, semg1, sems0, sems1, semm0, semm1, acc):
    core = lax.axis_index("c")
    sub = lax.axis_index("s")
    rows = [r0, r1, r2, r3]
    cols = [c0, c1, c2, c3]
    vals = [v0, v1, v2, v3]
    gbufs = [gb0, gb1]
    semg = [semg0, semg1]
    sems = [sems0, sems1]
    semm = [semm0, semm1]

    pltpu.sync_copy(zrows, acc.at[pl.ds(sub * ZPT, ZPT)])
    plsc.subcore_barrier()

    cbase = sub * (EPT // CHUNK)

    def meta_copies(i, slot, sem, make):
        f = pltpu.make_async_copy if make else pltpu.async_copy
        sl = pl.ds(cbase + i * GC, GC)
        return [f(adj3.at[0, sl], rows[slot], sem),
                f(adj3.at[1, sl], cols[slot], sem),
                f(val2.at[sl], vals[slot], sem)]

    def issue_gathers(ego, slot, p):
        return [pltpu.async_copy(
            ego.at[cols[slot].at[c]],
            gbufs[p].at[pl.ds(c * CHUNK, CHUNK)], semg[p])
            for c in range(GC)]

    def scale(slot, p):
        gbuf = gbufs[p]
        vv_ref = vals[slot]

        def body(i, _):
            c = i // (CHUNK // LANES)
            o = (i % (CHUNK // LANES)) * LANES
            vv = vv_ref[c, pl.ds(o, LANES)]
            for k in range(LANES):
                e = i * LANES + k
                vs = jnp.full((LANES,), vv[k], jnp.float32)
                for q in range(DH // LANES):
                    sl = pl.ds(q * LANES, LANES)
                    gbuf[e, sl] = gbuf[e, sl] * vs
            return 0
        lax.fori_loop(0, GROUP // LANES, body, 0, unroll=False)

    def run(ego, out):
        for cp in meta_copies(0, 0, semm[0], False):
            cp.wait()
        meta_copies(1, 1, semm[1], False)
        issue_gathers(ego, 0, 0)

        def outer(g4, _):
            for j in range(4):
                i = g4 * 4 + j
                p, q = j % 2, 1 - (j % 2)
                for c in range(GC):
                    pltpu.make_async_copy(
                        ego.at[cols[j].at[c]],
                        gbufs[p].at[pl.ds(c * CHUNK, CHUNK)],
                        semg[p]).wait()
                @pl.when(i + 2 < NGROUPS)
                def _():
                    meta_copies(i + 2, (j + 2) % 4, semm[p], False)
                @pl.when(i + 1 < NGROUPS)
                def _():
                    for cp in meta_copies(i + 1, (j + 1) % 4, semm[q], True):
                        cp.wait()

                @pl.when(i >= 1)
                def _():
                    for c in range(GC):
                        pltpu.make_async_copy(
                            gbufs[q].at[pl.ds(c * CHUNK, CHUNK)],
                            acc.at[rows[(j + 3) % 4].at[c]],
                            sems[q]).wait()

                @pl.when(i + 1 < NGROUPS)
                def _():
                    issue_gathers(ego, (j + 1) % 4, q)

                scale(j, p)
                for c in range(GC):
                    pltpu.async_copy(
                        gbufs[p].at[pl.ds(c * CHUNK, CHUNK)],
                        acc.at[rows[j].at[c]], sems[p], add=True)
            return 0

        lax.fori_loop(0, NGROUPS // 4, outer, 0, unroll=False)

        for c in range(GC):
            pltpu.make_async_copy(
                gbufs[1].at[pl.ds(c * CHUNK, CHUNK)],
                acc.at[rows[3].at[c]], sems[1]).wait()

        plsc.subcore_barrier()
        pltpu.sync_copy(acc.at[pl.ds(sub * ZPT, ZPT)],
                        out.at[pl.ds(sub * ZPT, ZPT)])

    @pl.when(core == 0)
    def _():
        run(ego_lo, out_lo)

    @pl.when(core == 1)
    def _():
        run(ego_hi, out_hi)


@jax.jit
def _spmm(ego_lo, ego_hi, adj3, val2):
    zrows = jnp.zeros((ZPT, DH), jnp.float32)
    mesh = plsc.VectorSubcoreMesh(core_axis_name="c", subcore_axis_name="s")
    half = jax.ShapeDtypeStruct((NP, DH), jnp.float32)
    idxbuf = pltpu.VMEM((GC, CHUNK), jnp.int32)
    valbuf = pltpu.VMEM((GC, CHUNK), jnp.float32)
    return pl.kernel(
        _spmm_body,
        out_type=(half, half),
        mesh=mesh,
        scratch_types=[
            idxbuf, idxbuf, idxbuf, idxbuf,
            idxbuf, idxbuf, idxbuf, idxbuf,
            valbuf, valbuf, valbuf, valbuf,
            pltpu.VMEM((GROUP, DH), jnp.float32),
            pltpu.VMEM((GROUP, DH), jnp.float32),
            pltpu.SemaphoreType.DMA,
            pltpu.SemaphoreType.DMA,
            pltpu.SemaphoreType.DMA,
            pltpu.SemaphoreType.DMA,
            pltpu.SemaphoreType.DMA,
            pltpu.SemaphoreType.DMA,
            pltpu.VMEM_SHARED((NP, DH), jnp.float32),
        ],
        compiler_params=pltpu.CompilerParams(use_tc_tiling_on_sc=False),
    )(ego_lo, ego_hi, adj3, val2, zrows)


def _dense_body(elo, ehi, slo, shi, wgc, bgc, wbi, bbi, new_lo, new_hi, norm):
    e = jnp.concatenate([elo[...], ehi[...]], axis=1)
    s = jnp.concatenate([slo[...], shi[...]], axis=1)
    x = (jnp.dot(s, wgc[...], preferred_element_type=jnp.float32) + bgc[...]
         + jnp.dot(e * s, wbi[...], preferred_element_type=jnp.float32)
         + bbi[...])
    h = jnp.where(x >= 0, x, 0.2 * x)
    n2 = jnp.sum(h * h, axis=1, keepdims=True)
    new_lo[...] = h[:, :DH]
    new_hi[...] = h[:, DH:]
    norm[...] = h / jnp.maximum(jnp.sqrt(n2), 1e-12)


@jax.jit
def _dense(elo, ehi, slo, shi, wgc, bgc, wbi, bbi):
    blk_h = pl.BlockSpec((TC_BLK, DH), lambda i: (i, 0))
    blk_f = pl.BlockSpec((TC_BLK, D), lambda i: (i, 0))
    full = pl.BlockSpec((D, D), lambda i: (0, 0))
    bias = pl.BlockSpec((1, D), lambda i: (0, 0))
    half = jax.ShapeDtypeStruct((NP, DH), jnp.float32)
    return pl.pallas_call(
        _dense_body,
        grid=(TC_GRID,),
        in_specs=[blk_h, blk_h, blk_h, blk_h, full, bias, full, bias],
        out_specs=[blk_h, blk_h, blk_f],
        out_shape=[half, half, jax.ShapeDtypeStruct((NP, D), jnp.float32)],
    )(elo, ehi, slo, shi, wgc, bgc, wbi, bbi)


def _lookup_body(s0, s1, s2, s3, idx, o0, o1, o2, o3, idxv, rbuf, sem):
    core = lax.axis_index("c")
    sub = lax.axis_index("s")
    wid = sub * NC + core
    base = wid * GPW
    pltpu.sync_copy(idx.at[pl.ds(base, GPW)], idxv)
    for slab, o in ((s0, o0), (s1, o1), (s2, o2), (s3, o3)):
        pltpu.async_copy(slab.at[idxv], rbuf, sem).wait()
        pltpu.sync_copy(rbuf, o.at[pl.ds(base, GPW)])


@jax.jit
def _lookup(s0, s1, s2, s3, idx):
    mesh = plsc.VectorSubcoreMesh(core_axis_name="c", subcore_axis_name="s")
    out = jax.ShapeDtypeStruct((GB, D), jnp.float32)
    return pl.kernel(
        _lookup_body,
        out_type=(out, out, out, out),
        mesh=mesh,
        scratch_types=[
            pltpu.VMEM((GPW,), jnp.int32),
            pltpu.VMEM((GPW, D), jnp.float32),
            pltpu.SemaphoreType.DMA,
        ],
        compiler_params=pltpu.CompilerParams(use_tc_tiling_on_sc=False),
    )(s0, s1, s2, s3, idx)


def kernel(user_emb, item_emb,
           W_gc_0, b_gc_0, W_bi_0, b_bi_0,
           W_gc_1, b_gc_1, W_bi_1, b_bi_1,
           W_gc_2, b_gc_2, W_bi_2, b_bi_2,
           adj_vals, adj_idx, users, pos_items, neg_items):
    Ws = [(W_gc_0, b_gc_0, W_bi_0, b_bi_0),
          (W_gc_1, b_gc_1, W_bi_1, b_bi_1),
          (W_gc_2, b_gc_2, W_bi_2, b_bi_2)]

    ego0 = jnp.zeros((NP, D), jnp.float32)
    ego0 = ego0.at[:N_USER].set(user_emb).at[N_USER:N].set(item_emb)

    adj3 = jnp.pad(adj_idx.astype(jnp.int32),
                   ((0, 0), (0, NNZ_P - NNZ))).reshape(2, -1, CHUNK)
    val2 = jnp.pad(adj_vals.astype(jnp.float32),
                   (0, NNZ_P - NNZ)).reshape(-1, CHUNK)

    slabs = [ego0]
    elo, ehi = ego0[:, :DH], ego0[:, DH:]
    for (wgc, bgc, wbi, bbi) in Ws:
        slo, shi = _spmm(elo, ehi, adj3, val2)
        elo, ehi, nrm = _dense(elo, ehi, slo, shi, wgc, bgc, wbi, bbi)
        slabs.append(nrm)

    idx = jnp.concatenate([users.astype(jnp.int32),
                           pos_items.astype(jnp.int32) + N_USER,
                           neg_items.astype(jnp.int32) + N_USER])
    g0, g1, g2, g3 = _lookup(slabs[0], slabs[1], slabs[2], slabs[3], idx)
    all_e = jnp.concatenate([g0, g1, g2, g3], axis=1)
    B = users.shape[0]
    return (all_e[:B], all_e[B:2 * B], all_e[2 * B:])


# speedup vs baseline: 8.4650x; 1.3102x over previous
import functools

import jax
import jax.numpy as jnp
from jax import lax
from jax.experimental import pallas as pl
from jax.experimental.pallas import tpu as pltpu
from jax.experimental.pallas import tpu_sc as plsc

N_USER = 25000
N_ITEM = 25000
N = N_USER + N_ITEM
D = 64
DH = D // 2
NNZ = 800000
LANES = 16

NC = 2
NS = 16
NW = NC * NS

NP = 50176
ZPT = NP // NS

NNZ_P = 811008
EPT = NNZ_P // NS
CHUNK = 128
GC = 3
GROUP = CHUNK * GC
NGROUPS = EPT // GROUP

F = 4
NPF = NP // F
DB = 448
TC_GRID = NPF // DB

GB = 3 * 1024
GPW = GB // NW


def _spmm_body(ego_lo, ego_hi, adj3, val2, zrows, out_lo, out_hi,
               r0, r1, r2, r3, c0, c1, c2, c3, v0, v1, v2, v3, gb0, gb1,
               semg0, semg1, sems0, sems1, semm0, semm1, acc):
    core = lax.axis_index("c")
    sub = lax.axis_index("s")
    rows = [r0, r1, r2, r3]
    cols = [c0, c1, c2, c3]
    vals = [v0, v1, v2, v3]
    gbufs = [gb0, gb1]
    semg = [semg0, semg1]
    sems = [sems0, sems1]
    semm = [semm0, semm1]

    pltpu.sync_copy(zrows, acc.at[pl.ds(sub * ZPT, ZPT)])
    plsc.subcore_barrier()

    cbase = sub * (EPT // CHUNK)

    def meta_copies(i, slot, sem, make):
        f = pltpu.make_async_copy if make else pltpu.async_copy
        sl = pl.ds(cbase + i * GC, GC)
        return [f(adj3.at[0, sl], rows[slot], sem),
                f(adj3.at[1, sl], cols[slot], sem),
                f(val2.at[sl], vals[slot], sem)]

    def issue_gathers(ego, slot, p):
        return [pltpu.async_copy(
            ego.at[cols[slot].at[c]],
            gbufs[p].at[pl.ds(c * CHUNK, CHUNK)], semg[p])
            for c in range(GC)]

    def scale(slot, p):
        gbuf = gbufs[p]
        vv_ref = vals[slot]

        def body(i, _):
            c = i // (CHUNK // LANES)
            o = (i % (CHUNK // LANES)) * LANES
            vv = vv_ref[c, pl.ds(o, LANES)]
            for k in range(LANES):
                e = i * LANES + k
                vs = jnp.full((LANES,), vv[k], jnp.float32)
                for q in range(DH // LANES):
                    sl = pl.ds(q * LANES, LANES)
                    gbuf[e, sl] = gbuf[e, sl] * vs
            return 0
        lax.fori_loop(0, GROUP // LANES, body, 0, unroll=False)

    def run(ego, out):
        for cp in meta_copies(0, 0, semm[0], False):
            cp.wait()
        meta_copies(1, 1, semm[1], False)
        issue_gathers(ego, 0, 0)

        def outer(g4, _):
            for j in range(4):
                i = g4 * 4 + j
                p, q = j % 2, 1 - (j % 2)
                for c in range(GC):
                    pltpu.make_async_copy(
                        ego.at[cols[j].at[c]],
                        gbufs[p].at[pl.ds(c * CHUNK, CHUNK)],
                        semg[p]).wait()
                @pl.when(i + 2 < NGROUPS)
                def _():
                    meta_copies(i + 2, (j + 2) % 4, semm[p], False)
                @pl.when(i + 1 < NGROUPS)
                def _():
                    for cp in meta_copies(i + 1, (j + 1) % 4, semm[q], True):
                        cp.wait()

                @pl.when(i >= 1)
                def _():
                    for c in range(GC):
                        pltpu.make_async_copy(
                            gbufs[q].at[pl.ds(c * CHUNK, CHUNK)],
                            acc.at[rows[(j + 3) % 4].at[c]],
                            sems[q]).wait()

                @pl.when(i + 1 < NGROUPS)
                def _():
                    issue_gathers(ego, (j + 1) % 4, q)

                scale(j, p)
                for c in range(GC):
                    pltpu.async_copy(
                        gbufs[p].at[pl.ds(c * CHUNK, CHUNK)],
                        acc.at[rows[j].at[c]], sems[p], add=True)
            return 0

        lax.fori_loop(0, NGROUPS // 4, outer, 0, unroll=False)

        for c in range(GC):
            pltpu.make_async_copy(
                gbufs[1].at[pl.ds(c * CHUNK, CHUNK)],
                acc.at[rows[3].at[c]], sems[1]).wait()

        plsc.subcore_barrier()
        pltpu.sync_copy(acc.at[pl.ds(sub * ZPT, ZPT)],
                        out.at[pl.ds(sub * ZPT, ZPT)])

    @pl.when(core == 0)
    def _():
        run(ego_lo, out_lo)

    @pl.when(core == 1)
    def _():
        run(ego_hi, out_hi)


@jax.jit
def _spmm(ego_lo, ego_hi, adj3, val2):
    zrows = jnp.zeros((ZPT, DH), jnp.float32)
    mesh = plsc.VectorSubcoreMesh(core_axis_name="c", subcore_axis_name="s")
    half = jax.ShapeDtypeStruct((NP, DH), jnp.float32)
    idxbuf = pltpu.VMEM((GC, CHUNK), jnp.int32)
    valbuf = pltpu.VMEM((GC, CHUNK), jnp.float32)
    return pl.kernel(
        _spmm_body,
        out_type=(half, half),
        mesh=mesh,
        scratch_types=[
            idxbuf, idxbuf, idxbuf, idxbuf,
            idxbuf, idxbuf, idxbuf, idxbuf,
            valbuf, valbuf, valbuf, valbuf,
            pltpu.VMEM((GROUP, DH), jnp.float32),
            pltpu.VMEM((GROUP, DH), jnp.float32),
            pltpu.SemaphoreType.DMA,
            pltpu.SemaphoreType.DMA,
            pltpu.SemaphoreType.DMA,
            pltpu.SemaphoreType.DMA,
            pltpu.SemaphoreType.DMA,
            pltpu.SemaphoreType.DMA,
            pltpu.VMEM_SHARED((NP, DH), jnp.float32),
        ],
        compiler_params=pltpu.CompilerParams(use_tc_tiling_on_sc=False),
    )(ego_lo, ego_hi, adj3, val2, zrows)


def _dense_body(elo, ehi, slo, shi, wgl, wgh, bg4, wbl, wbh, bb4, ones4,
                new_lo, new_hi, norm):
    e_lo, e_hi = elo[...], ehi[...]
    s_lo, s_hi = slo[...], shi[...]
    x = (jnp.dot(s_lo, wgl[...], preferred_element_type=jnp.float32)
         + jnp.dot(s_hi, wgh[...], preferred_element_type=jnp.float32)
         + jnp.dot(e_lo * s_lo, wbl[...], preferred_element_type=jnp.float32)
         + jnp.dot(e_hi * s_hi, wbh[...], preferred_element_type=jnp.float32)
         + bg4[...] + bb4[...])
    h = jnp.where(x >= 0, x, 0.2 * x)
    n2 = jnp.dot(h * h, ones4[...], preferred_element_type=jnp.float32)
    new_lo[...] = jnp.concatenate(
        [h[:, k * D:k * D + DH] for k in range(F)], axis=1)
    new_hi[...] = jnp.concatenate(
        [h[:, k * D + DH:(k + 1) * D] for k in range(F)], axis=1)
    norm[...] = h / jnp.maximum(jnp.sqrt(n2), 1e-12)


@jax.jit
def _dense(elo, ehi, slo, shi, wgl, wgh, bg4, wbl, wbh, bb4, ones4):
    blk = pl.BlockSpec((DB, 128), lambda i: (i, 0))
    blk_w = pl.BlockSpec((DB, F * D), lambda i: (i, 0))
    wspec = pl.BlockSpec((128, F * D), lambda i: (0, 0))
    ospec = pl.BlockSpec((F * D, F * D), lambda i: (0, 0))
    bspec = pl.BlockSpec((1, F * D), lambda i: (0, 0))
    fold = jax.ShapeDtypeStruct((NPF, 128), jnp.float32)
    return pl.pallas_call(
        _dense_body,
        grid=(TC_GRID,),
        in_specs=[blk, blk, blk, blk,
                  wspec, wspec, bspec, wspec, wspec, bspec, ospec],
        out_specs=[blk, blk, blk_w],
        out_shape=[fold, fold,
                   jax.ShapeDtypeStruct((NPF, F * D), jnp.float32)],
    )(elo, ehi, slo, shi, wgl, wgh, bg4, wbl, wbh, bb4, ones4)


def _lookup_body(s0, s1, s2, s3, idx, o0, o1, o2, o3, idxv, rbuf, sem):
    core = lax.axis_index("c")
    sub = lax.axis_index("s")
    wid = sub * NC + core
    base = wid * GPW
    pltpu.sync_copy(idx.at[pl.ds(base, GPW)], idxv)
    for slab, o in ((s0, o0), (s1, o1), (s2, o2), (s3, o3)):
        pltpu.async_copy(slab.at[idxv], rbuf, sem).wait()
        pltpu.sync_copy(rbuf, o.at[pl.ds(base, GPW)])


@jax.jit
def _lookup(s0, s1, s2, s3, idx):
    mesh = plsc.VectorSubcoreMesh(core_axis_name="c", subcore_axis_name="s")
    out = jax.ShapeDtypeStruct((GB, D), jnp.float32)
    return pl.kernel(
        _lookup_body,
        out_type=(out, out, out, out),
        mesh=mesh,
        scratch_types=[
            pltpu.VMEM((GPW,), jnp.int32),
            pltpu.VMEM((GPW, D), jnp.float32),
            pltpu.SemaphoreType.DMA,
        ],
        compiler_params=pltpu.CompilerParams(use_tc_tiling_on_sc=False),
    )(s0, s1, s2, s3, idx)


def kernel(user_emb, item_emb,
           W_gc_0, b_gc_0, W_bi_0, b_bi_0,
           W_gc_1, b_gc_1, W_bi_1, b_bi_1,
           W_gc_2, b_gc_2, W_bi_2, b_bi_2,
           adj_vals, adj_idx, users, pos_items, neg_items):
    Ws = [(W_gc_0, b_gc_0, W_bi_0, b_bi_0),
          (W_gc_1, b_gc_1, W_bi_1, b_bi_1),
          (W_gc_2, b_gc_2, W_bi_2, b_bi_2)]

    ego0 = jnp.zeros((NP, D), jnp.float32)
    ego0 = ego0.at[:N_USER].set(user_emb).at[N_USER:N].set(item_emb)

    adj3 = jnp.pad(adj_idx.astype(jnp.int32),
                   ((0, 0), (0, NNZ_P - NNZ))).reshape(2, -1, CHUNK)
    val2 = jnp.pad(adj_vals.astype(jnp.float32),
                   (0, NNZ_P - NNZ)).reshape(-1, CHUNK)

    eye4 = jnp.eye(F, dtype=jnp.float32)
    ones4 = jnp.kron(eye4, jnp.ones((D, D), jnp.float32))

    slabs = [ego0]
    elo, ehi = ego0[:, :DH], ego0[:, DH:]
    for (wgc, bgc, wbi, bbi) in Ws:
        slo, shi = _spmm(elo, ehi, adj3, val2)
        wgl = jnp.kron(eye4, wgc[:DH])
        wgh = jnp.kron(eye4, wgc[DH:])
        wbl = jnp.kron(eye4, wbi[:DH])
        wbh = jnp.kron(eye4, wbi[DH:])
        bg4 = jnp.tile(bgc, (1, F))
        bb4 = jnp.tile(bbi, (1, F))
        nlo, nhi, nrm = _dense(elo.reshape(NPF, 128), ehi.reshape(NPF, 128),
                               slo.reshape(NPF, 128), shi.reshape(NPF, 128),
                               wgl, wgh, bg4, wbl, wbh, bb4, ones4)
        elo = nlo.reshape(NP, DH)
        ehi = nhi.reshape(NP, DH)
        slabs.append(nrm.reshape(NP, D))

    idx = jnp.concatenate([users.astype(jnp.int32),
                           pos_items.astype(jnp.int32) + N_USER,
                           neg_items.astype(jnp.int32) + N_USER])
    g0, g1, g2, g3 = _lookup(slabs[0], slabs[1], slabs[2], slabs[3], idx)
    all_e = jnp.concatenate([g0, g1, g2, g3], axis=1)
    B = users.shape[0]
    return (all_e[:B], all_e[B:2 * B], all_e[2 * B:])
